# Initial kernel scaffold; baseline (speedup 1.0000x reference)
import functools

import jax
import jax.numpy as jnp
from jax import lax
from jax.experimental import pallas as pl
from jax.experimental.pallas import tpu as pltpu
from jax.experimental.pallas import tpu_sc as plsc

N = 10000
E = 320000
NSUB = 16
NCORE = 2
NW = NCORE * NSUB
CHUNK = 128
EPW = 10240
E_PAD = NW * EPW
CH = EPW // CHUNK
N_ACC = 10240
ZROWS = N_ACC // NSUB

_PREC = jax.lax.Precision.HIGHEST


def _mesh():
    return plsc.VectorSubcoreMesh(core_axis_name="c", subcore_axis_name="s")


def _deg_partials(dst3, ones_rows, zrows):
    @functools.partial(
        pl.kernel,
        out_type=jax.ShapeDtypeStruct((NCORE, N_ACC, 8), jnp.float32),
        mesh=_mesh(),
        scratch_types=[
            pltpu.VMEM((CH, CHUNK), jnp.int32),
            pltpu.VMEM((CHUNK, 8), jnp.float32),
            pltpu.VMEM_SHARED((N_ACC, 8), jnp.float32),
            pltpu.SemaphoreType.DMA,
        ],
    )
    def deg_kernel(dst_hbm, ones_hbm, z_hbm, out_hbm, dstv, onesv, acc, sem):
        c = lax.axis_index("c")
        s = lax.axis_index("s")
        w = c * NSUB + s
        pltpu.sync_copy(z_hbm, acc.at[pl.ds(s * ZROWS, ZROWS)])
        pltpu.sync_copy(dst_hbm.at[w], dstv)
        pltpu.sync_copy(ones_hbm, onesv)
        plsc.subcore_barrier()

        @pl.loop(0, CH)
        def _(j):
            pltpu.sync_copy(onesv, acc.at[dstv.at[j]], add=True)

        plsc.subcore_barrier()
        pltpu.sync_copy(acc.at[pl.ds(s * ZROWS, ZROWS)],
                        out_hbm.at[c, pl.ds(s * ZROWS, ZROWS)])

    return deg_kernel(dst3, ones_rows, zrows)


def _hop_partials(F, u, src3, dst3, zrows):
    @functools.partial(
        pl.kernel,
        out_type=jax.ShapeDtypeStruct((NCORE, N_ACC, F), jnp.float32),
        mesh=_mesh(),
        scratch_types=[
            pltpu.VMEM((CH, CHUNK), jnp.int32),
            pltpu.VMEM((CH, CHUNK), jnp.int32),
            pltpu.VMEM((CHUNK, F), jnp.float32),
            pltpu.VMEM((CHUNK, F), jnp.float32),
            pltpu.VMEM_SHARED((N_ACC, F), jnp.float32),
            pltpu.SemaphoreType.DMA,
            pltpu.SemaphoreType.DMA,
        ],
    )
    def hop_kernel(u_hbm, src_hbm, dst_hbm, z_hbm, out_hbm,
                   srcv, dstv, rows0, rows1, acc, sem0, sem1):
        c = lax.axis_index("c")
        s = lax.axis_index("s")
        w = c * NSUB + s
        pltpu.sync_copy(z_hbm, acc.at[pl.ds(s * ZROWS, ZROWS)])
        pltpu.sync_copy(src_hbm.at[w], srcv)
        pltpu.sync_copy(dst_hbm.at[w], dstv)
        plsc.subcore_barrier()

        pltpu.async_copy(u_hbm.at[srcv.at[0]], rows0, sem0)

        @pl.loop(0, CH, step=2)
        def _(j):
            pltpu.async_copy(u_hbm.at[srcv.at[j + 1]], rows1, sem1)
            pltpu.make_async_copy(u_hbm.at[srcv.at[j]], rows0, sem0).wait()
            pltpu.sync_copy(rows0, acc.at[dstv.at[j]], add=True)

            @pl.when(j + 2 < CH)
            def _():
                pltpu.async_copy(u_hbm.at[srcv.at[j + 2]], rows0, sem0)

            pltpu.make_async_copy(u_hbm.at[srcv.at[j + 1]], rows1, sem1).wait()
            pltpu.sync_copy(rows1, acc.at[dstv.at[j + 1]], add=True)

        plsc.subcore_barrier()
        pltpu.sync_copy(acc.at[pl.ds(s * ZROWS, ZROWS)],
                        out_hbm.at[c, pl.ds(s * ZROWS, ZROWS)])

    return hop_kernel(u, src3, dst3, zrows)


def _tc_matmul(h, Wp):
    M, _ = h.shape
    _, Fo = Wp.shape

    def body(h_ref, w_ref, o_ref):
        o_ref[...] = jnp.dot(h_ref[...], w_ref[...],
                             preferred_element_type=jnp.float32,
                             precision=_PREC)

    return pl.pallas_call(
        body, out_shape=jax.ShapeDtypeStruct((M, Fo), jnp.float32))(h, Wp)


def _tc_pre(degp, P1):
    def body(d_ref, p_ref, dis_ref, u_ref):
        deg = d_ref[0, :N, 0:1] + d_ref[1, :N, 0:1]
        dis = jnp.where(deg > 0.0,
                        lax.rsqrt(jnp.maximum(deg, 1e-12)),
                        0.0)
        dis_ref[...] = dis
        u_ref[...] = dis * p_ref[:, 96:128]

    return pl.pallas_call(
        body,
        out_shape=(jax.ShapeDtypeStruct((N, 1), jnp.float32),
                   jax.ShapeDtypeStruct((N, 32), jnp.float32)),
    )(degp, P1)


def _tc_mid(accs, P, dis, k, F):
    def body(a_ref, p_ref, d_ref, u_ref):
        raw = a_ref[0, :N, :] + a_ref[1, :N, :]
        dis = d_ref[...]
        u_ref[...] = dis * p_ref[:, k * F:(k + 1) * F] + (dis * dis) * raw

    return pl.pallas_call(
        body, out_shape=jax.ShapeDtypeStruct((N, F), jnp.float32),
    )(accs, P, dis)


def _tc_layer(accs, P, dis, b, Wnext, F, Fn):
    _, Fo = Wnext.shape

    def body(a_ref, p_ref, d_ref, b_ref, w_ref, pn_ref, u_ref):
        raw = a_ref[0, :N, :] + a_ref[1, :N, :]
        dis = d_ref[...]
        t = p_ref[:, 0:F] + dis * raw
        h = jnp.maximum(t + b_ref[...], 0.0)
        pn = jnp.dot(h, w_ref[...], preferred_element_type=jnp.float32,
                     precision=_PREC)
        pn_ref[...] = pn
        u_ref[...] = dis * pn[:, 3 * Fn:4 * Fn]

    return pl.pallas_call(
        body,
        out_shape=(jax.ShapeDtypeStruct((N, Fo), jnp.float32),
                   jax.ShapeDtypeStruct((N, Fn), jnp.float32)),
    )(accs, P, dis, b.reshape(1, -1), Wnext)


def _tc_layer23(accs, P2, dis, b2, W3p):
    def body(a_ref, p_ref, d_ref, b_ref, w_ref, pn_ref, u_ref):
        raw = a_ref[0, :N, :] + a_ref[1, :N, :]
        dis = d_ref[...]
        t = p_ref[:, 0:16] + dis * raw
        h = jnp.maximum(t + b_ref[...], 0.0)
        pn = jnp.dot(h, w_ref[...], preferred_element_type=jnp.float32,
                     precision=_PREC)
        pn_ref[...] = pn
        u_ref[...] = jnp.concatenate(
            [dis * pn[:, 6:8], jnp.zeros((N, 6), jnp.float32)], axis=1)

    return pl.pallas_call(
        body,
        out_shape=(jax.ShapeDtypeStruct((N, 8), jnp.float32),
                   jax.ShapeDtypeStruct((N, 8), jnp.float32)),
    )(accs, P2, dis, b2.reshape(1, -1), W3p)


def _tc_mid3(accs, P3, dis, k):
    def body(a_ref, p_ref, d_ref, u_ref):
        raw = a_ref[0, :N, 0:2] + a_ref[1, :N, 0:2]
        dis = d_ref[...]
        u2 = dis * p_ref[:, 2 * k:2 * k + 2] + (dis * dis) * raw
        u_ref[...] = jnp.concatenate(
            [u2, jnp.zeros((N, 6), jnp.float32)], axis=1)

    return pl.pallas_call(
        body, out_shape=jax.ShapeDtypeStruct((N, 8), jnp.float32),
    )(accs, P3, dis)


def _tc_final(accs, P3, dis, b3):
    def body(a_ref, p_ref, d_ref, b_ref, o_ref):
        raw = a_ref[0, :N, 0:2] + a_ref[1, :N, 0:2]
        z = p_ref[:, 0:2] + d_ref[...] * raw + b_ref[...]
        m = jnp.max(z, axis=1, keepdims=True)
        lse = m + jnp.log(jnp.sum(jnp.exp(z - m), axis=1, keepdims=True))
        o_ref[...] = z - lse

    return pl.pallas_call(
        body, out_shape=jax.ShapeDtypeStruct((N, 2), jnp.float32),
    )(accs, P3, dis, b3.reshape(1, -1))


def kernel(x, edge_index, W1, b1, W2, b2, W3, b3):
    src = edge_index[0].astype(jnp.int32)
    dst = edge_index[1].astype(jnp.int32)
    pad = E_PAD - E
    src3 = jnp.concatenate([src, jnp.zeros((pad,), jnp.int32)]).reshape(
        NW, CH, CHUNK)
    dst3 = jnp.concatenate([dst, jnp.full((pad,), N, jnp.int32)]).reshape(
        NW, CH, CHUNK)

    ones8 = jnp.ones((CHUNK, 8), jnp.float32)
    z8 = jnp.zeros((ZROWS, 8), jnp.float32)
    z16 = jnp.zeros((ZROWS, 16), jnp.float32)
    z32 = jnp.zeros((ZROWS, 32), jnp.float32)

    W1p = jnp.concatenate([W1[i * 128:(i + 1) * 128] for i in range(4)], axis=1)
    W2p = jnp.concatenate([W2[i * 32:(i + 1) * 32] for i in range(4)], axis=1)
    W3p = jnp.concatenate([W3[i * 16:(i + 1) * 16] for i in range(4)], axis=1)

    degp = _deg_partials(dst3, ones8, z8)
    P1 = _tc_matmul(x, W1p)
    dis, u = _tc_pre(degp, P1)

    for k in (2, 1):
        accs = _hop_partials(32, u, src3, dst3, z32)
        u = _tc_mid(accs, P1, dis, k, 32)
    accs = _hop_partials(32, u, src3, dst3, z32)
    P2, u = _tc_layer(accs, P1, dis, b1, W2p, 32, 16)

    for k in (2, 1):
        accs = _hop_partials(16, u, src3, dst3, z16)
        u = _tc_mid(accs, P2, dis, k, 16)
    accs = _hop_partials(16, u, src3, dst3, z16)
    P3, u = _tc_layer23(accs, P2, dis, b2, W3p)

    for k in (2, 1):
        accs = _hop_partials(8, u, src3, dst3, z8)
        u = _tc_mid3(accs, P3, dis, k)
    accs = _hop_partials(8, u, src3, dst3, z8)
    return _tc_final(accs, P3, dis, b3)


# speedup vs baseline: 15.0487x; 15.0487x over previous
import functools

import jax
import jax.numpy as jnp
from jax import lax
from jax.experimental import pallas as pl
from jax.experimental.pallas import tpu as pltpu
from jax.experimental.pallas import tpu_sc as plsc

N = 10000
E = 320000
NSUB = 16
NCORE = 2
NW = NCORE * NSUB
CHUNK = 128
EPW = 10240
E_PAD = NW * EPW
CH = EPW // CHUNK
N_ACC = 10240
ZROWS = N_ACC // NSUB

_PREC = jax.lax.Precision.HIGHEST


def _mesh():
    return plsc.VectorSubcoreMesh(core_axis_name="c", subcore_axis_name="s")


_SC_PARAMS = pltpu.CompilerParams(use_tc_tiling_on_sc=False)


def _deg_partials(dst3, ones_rows, zrows):
    @functools.partial(
        pl.kernel,
        out_type=jax.ShapeDtypeStruct((NCORE, N_ACC, 8), jnp.float32),
        mesh=_mesh(),
        scratch_types=[
            pltpu.VMEM((CH, CHUNK), jnp.int32),
            pltpu.VMEM((CHUNK, 8), jnp.float32),
            pltpu.VMEM_SHARED((N_ACC, 8), jnp.float32),
            pltpu.SemaphoreType.DMA,
        ],
        compiler_params=_SC_PARAMS,
    )
    def deg_kernel(dst_hbm, ones_hbm, z_hbm, out_hbm, dstv, onesv, acc, sem):
        c = lax.axis_index("c")
        s = lax.axis_index("s")
        w = c * NSUB + s
        pltpu.sync_copy(z_hbm, acc.at[pl.ds(s * ZROWS, ZROWS)])
        pltpu.sync_copy(dst_hbm.at[w], dstv)
        pltpu.sync_copy(ones_hbm, onesv)
        plsc.subcore_barrier()

        @pl.loop(0, CH)
        def _(j):
            pltpu.sync_copy(onesv, acc.at[dstv.at[j]], add=True)

        plsc.subcore_barrier()
        pltpu.sync_copy(acc.at[pl.ds(s * ZROWS, ZROWS)],
                        out_hbm.at[c, pl.ds(s * ZROWS, ZROWS)])

    return deg_kernel(dst3, ones_rows, zrows)


def _hop_partials(F, u, src3, dst3, zrows):
    @functools.partial(
        pl.kernel,
        out_type=jax.ShapeDtypeStruct((NCORE, N_ACC, F), jnp.float32),
        mesh=_mesh(),
        scratch_types=[
            pltpu.VMEM((CH, CHUNK), jnp.int32),
            pltpu.VMEM((CH, CHUNK), jnp.int32),
            pltpu.VMEM((CHUNK, F), jnp.float32),
            pltpu.VMEM((CHUNK, F), jnp.float32),
            pltpu.VMEM_SHARED((N_ACC, F), jnp.float32),
            pltpu.SemaphoreType.DMA,
            pltpu.SemaphoreType.DMA,
        ],
        compiler_params=_SC_PARAMS,
    )
    def hop_kernel(u_hbm, src_hbm, dst_hbm, z_hbm, out_hbm,
                   srcv, dstv, rows0, rows1, acc, sem0, sem1):
        c = lax.axis_index("c")
        s = lax.axis_index("s")
        w = c * NSUB + s
        pltpu.sync_copy(z_hbm, acc.at[pl.ds(s * ZROWS, ZROWS)])
        pltpu.sync_copy(src_hbm.at[w], srcv)
        pltpu.sync_copy(dst_hbm.at[w], dstv)
        plsc.subcore_barrier()

        pltpu.async_copy(u_hbm.at[srcv.at[0]], rows0, sem0)

        @pl.loop(0, CH, step=2)
        def _(j):
            pltpu.async_copy(u_hbm.at[srcv.at[j + 1]], rows1, sem1)
            pltpu.make_async_copy(u_hbm.at[srcv.at[j]], rows0, sem0).wait()
            pltpu.sync_copy(rows0, acc.at[dstv.at[j]], add=True)

            @pl.when(j + 2 < CH)
            def _():
                pltpu.async_copy(u_hbm.at[srcv.at[j + 2]], rows0, sem0)

            pltpu.make_async_copy(u_hbm.at[srcv.at[j + 1]], rows1, sem1).wait()
            pltpu.sync_copy(rows1, acc.at[dstv.at[j + 1]], add=True)

        plsc.subcore_barrier()
        pltpu.sync_copy(acc.at[pl.ds(s * ZROWS, ZROWS)],
                        out_hbm.at[c, pl.ds(s * ZROWS, ZROWS)])

    return hop_kernel(u, src3, dst3, zrows)


def _tc_matmul(h, Wp):
    M, _ = h.shape
    _, Fo = Wp.shape

    def body(h_ref, w_ref, o_ref):
        o_ref[...] = jnp.dot(h_ref[...], w_ref[...],
                             preferred_element_type=jnp.float32,
                             precision=_PREC)

    return pl.pallas_call(
        body, out_shape=jax.ShapeDtypeStruct((M, Fo), jnp.float32))(h, Wp)


def _tc_pre(degp, P1):
    def body(d_ref, p_ref, dis_ref, u_ref):
        deg = d_ref[0, :N, 0:1] + d_ref[1, :N, 0:1]
        dis = jnp.where(deg > 0.0,
                        lax.rsqrt(jnp.maximum(deg, 1e-12)),
                        0.0)
        dis_ref[...] = dis
        u_ref[...] = dis * p_ref[:, 96:128]

    return pl.pallas_call(
        body,
        out_shape=(jax.ShapeDtypeStruct((N, 1), jnp.float32),
                   jax.ShapeDtypeStruct((N, 32), jnp.float32)),
    )(degp, P1)


def _tc_mid(accs, P, dis, k, F):
    def body(a_ref, p_ref, d_ref, u_ref):
        raw = a_ref[0, :N, :] + a_ref[1, :N, :]
        dis = d_ref[...]
        u_ref[...] = dis * p_ref[:, k * F:(k + 1) * F] + (dis * dis) * raw

    return pl.pallas_call(
        body, out_shape=jax.ShapeDtypeStruct((N, F), jnp.float32),
    )(accs, P, dis)


def _tc_layer(accs, P, dis, b, Wnext, F, Fn):
    _, Fo = Wnext.shape

    def body(a_ref, p_ref, d_ref, b_ref, w_ref, pn_ref, u_ref):
        raw = a_ref[0, :N, :] + a_ref[1, :N, :]
        dis = d_ref[...]
        t = p_ref[:, 0:F] + dis * raw
        h = jnp.maximum(t + b_ref[...], 0.0)
        pn = jnp.dot(h, w_ref[...], preferred_element_type=jnp.float32,
                     precision=_PREC)
        pn_ref[...] = pn
        u_ref[...] = dis * pn[:, 3 * Fn:4 * Fn]

    return pl.pallas_call(
        body,
        out_shape=(jax.ShapeDtypeStruct((N, Fo), jnp.float32),
                   jax.ShapeDtypeStruct((N, Fn), jnp.float32)),
    )(accs, P, dis, b.reshape(1, -1), Wnext)


def _tc_layer23(accs, P2, dis, b2, W3p):
    def body(a_ref, p_ref, d_ref, b_ref, w_ref, pn_ref, u_ref):
        raw = a_ref[0, :N, :] + a_ref[1, :N, :]
        dis = d_ref[...]
        t = p_ref[:, 0:16] + dis * raw
        h = jnp.maximum(t + b_ref[...], 0.0)
        pn = jnp.dot(h, w_ref[...], preferred_element_type=jnp.float32,
                     precision=_PREC)
        pn_ref[...] = pn
        u_ref[...] = jnp.concatenate(
            [dis * pn[:, 6:8], jnp.zeros((N, 6), jnp.float32)], axis=1)

    return pl.pallas_call(
        body,
        out_shape=(jax.ShapeDtypeStruct((N, 8), jnp.float32),
                   jax.ShapeDtypeStruct((N, 8), jnp.float32)),
    )(accs, P2, dis, b2.reshape(1, -1), W3p)


def _tc_mid3(accs, P3, dis, k):
    def body(a_ref, p_ref, d_ref, u_ref):
        raw = a_ref[0, :N, 0:2] + a_ref[1, :N, 0:2]
        dis = d_ref[...]
        u2 = dis * p_ref[:, 2 * k:2 * k + 2] + (dis * dis) * raw
        u_ref[...] = jnp.concatenate(
            [u2, jnp.zeros((N, 6), jnp.float32)], axis=1)

    return pl.pallas_call(
        body, out_shape=jax.ShapeDtypeStruct((N, 8), jnp.float32),
    )(accs, P3, dis)


def _tc_final(accs, P3, dis, b3):
    def body(a_ref, p_ref, d_ref, b_ref, o_ref):
        raw = a_ref[0, :N, 0:2] + a_ref[1, :N, 0:2]
        z = p_ref[:, 0:2] + d_ref[...] * raw + b_ref[...]
        m = jnp.max(z, axis=1, keepdims=True)
        lse = m + jnp.log(jnp.sum(jnp.exp(z - m), axis=1, keepdims=True))
        o_ref[...] = z - lse

    return pl.pallas_call(
        body, out_shape=jax.ShapeDtypeStruct((N, 2), jnp.float32),
    )(accs, P3, dis, b3.reshape(1, -1))


def kernel(x, edge_index, W1, b1, W2, b2, W3, b3):
    src = edge_index[0].astype(jnp.int32)
    dst = edge_index[1].astype(jnp.int32)
    pad = E_PAD - E
    src3 = jnp.concatenate([src, jnp.zeros((pad,), jnp.int32)]).reshape(
        NW, CH, CHUNK)
    dst3 = jnp.concatenate([dst, jnp.full((pad,), N, jnp.int32)]).reshape(
        NW, CH, CHUNK)

    ones8 = jnp.ones((CHUNK, 8), jnp.float32)
    z8 = jnp.zeros((ZROWS, 8), jnp.float32)
    z16 = jnp.zeros((ZROWS, 16), jnp.float32)
    z32 = jnp.zeros((ZROWS, 32), jnp.float32)

    W1p = jnp.concatenate([W1[i * 128:(i + 1) * 128] for i in range(4)], axis=1)
    W2p = jnp.concatenate([W2[i * 32:(i + 1) * 32] for i in range(4)], axis=1)
    W3p = jnp.concatenate([W3[i * 16:(i + 1) * 16] for i in range(4)], axis=1)

    degp = _deg_partials(dst3, ones8, z8)
    P1 = _tc_matmul(x, W1p)
    dis, u = _tc_pre(degp, P1)

    for k in (2, 1):
        accs = _hop_partials(32, u, src3, dst3, z32)
        u = _tc_mid(accs, P1, dis, k, 32)
    accs = _hop_partials(32, u, src3, dst3, z32)
    P2, u = _tc_layer(accs, P1, dis, b1, W2p, 32, 16)

    for k in (2, 1):
        accs = _hop_partials(16, u, src3, dst3, z16)
        u = _tc_mid(accs, P2, dis, k, 16)
    accs = _hop_partials(16, u, src3, dst3, z16)
    P3, u = _tc_layer23(accs, P2, dis, b2, W3p)

    for k in (2, 1):
        accs = _hop_partials(8, u, src3, dst3, z8)
        u = _tc_mid3(accs, P3, dis, k)
    accs = _hop_partials(8, u, src3, dst3, z8)
    return _tc_final(accs, P3, dis, b3)
